# baseline (speedup 1.0000x reference)
import functools

import jax
import jax.numpy as jnp
from jax import lax
from jax.experimental import pallas as pl
from jax.experimental.pallas import tpu as pltpu
from jax.experimental.pallas import tpu_sc as plsc

VOCAB = 8192
D = 8192
SPLIT = 2
D2 = D // SPLIT
NC = 2
NS = 16
NW = NC * NS
K = 8


def _make_gather(b2):
    pw = b2 // NW
    chunks = pw // K
    mesh = plsc.VectorSubcoreMesh(core_axis_name="c", subcore_axis_name="s")

    @functools.partial(
        pl.kernel,
        mesh=mesh,
        out_type=jax.ShapeDtypeStruct((b2, D2), jnp.float32),
        scratch_types=[
            pltpu.VMEM((pw,), jnp.int32),
            pltpu.VMEM((K, D2), jnp.float32),
            pltpu.SemaphoreType.DMA,
        ],
    )
    def gather_kernel(table_hbm, idx_hbm, out_hbm, idx_v, rows_v, sem):
        cid = lax.axis_index("c")
        sid = lax.axis_index("s")
        wid = sid * NC + cid
        base = wid * pw
        pltpu.sync_copy(idx_hbm.at[pl.ds(base, pw)], idx_v)

        def chunk_body(i, carry):
            off = i * K
            pltpu.async_copy(
                table_hbm.at[idx_v.at[pl.ds(off, K)]], rows_v, sem
            ).wait()
            pltpu.sync_copy(rows_v, out_hbm.at[pl.ds(base + off, K)])
            return carry

        lax.fori_loop(0, chunks, chunk_body, 0)

    return gather_kernel


def kernel(idx, logits_table):
    b, t = idx.shape
    table2 = logits_table.reshape(VOCAB * SPLIT, D2)
    idxf = idx.reshape(-1).astype(jnp.int32)
    idx2 = (idxf[:, None] * SPLIT
            + jnp.arange(SPLIT, dtype=jnp.int32)[None, :]).reshape(-1)
    out2 = _make_gather(idx2.shape[0])(table2, idx2)
    return out2.reshape(b, t, D)


# speedup vs baseline: 1.0253x; 1.0253x over previous
import functools

import jax
import jax.numpy as jnp
from jax import lax
from jax.experimental import pallas as pl
from jax.experimental.pallas import tpu as pltpu
from jax.experimental.pallas import tpu_sc as plsc

VOCAB = 8192
D = 8192
SPLIT = 4
D2 = D // SPLIT
NC = 2
NS = 16
NW = NC * NS
K = 8
NG = 2


def _make_gather(b2):
    pw = b2 // NW
    chunks = pw // K
    rounds = chunks // NG
    pairs = rounds // 2
    mesh = plsc.VectorSubcoreMesh(core_axis_name="c", subcore_axis_name="s")

    @functools.partial(
        pl.kernel,
        mesh=mesh,
        out_type=jax.ShapeDtypeStruct((b2, D2), jnp.float32),
        scratch_types=[
            pltpu.VMEM((pw,), jnp.int32),
            pltpu.VMEM((2 * NG * K, D2), jnp.float32),
            [pltpu.SemaphoreType.DMA] * (2 * NG),
            [pltpu.SemaphoreType.DMA] * (2 * NG),
        ],
    )
    def gather_kernel(table_hbm, idx_hbm, out_hbm, idx_v, rows_v, gsems, ssems):
        cid = lax.axis_index("c")
        sid = lax.axis_index("s")
        wid = sid * NC + cid
        base = wid * pw
        pltpu.sync_copy(idx_hbm.at[pl.ds(base, pw)], idx_v)

        def buf(g, b):
            return rows_v.at[pl.ds((g * NG + b) * K, K)]

        def g_copy(g, b, c):
            return pltpu.make_async_copy(
                table_hbm.at[idx_v.at[pl.ds(c * K, K)]],
                buf(g, b),
                gsems[g * NG + b],
            )

        def s_copy(g, b, c):
            return pltpu.make_async_copy(
                buf(g, b),
                out_hbm.at[pl.ds(base + c * K, K)],
                ssems[g * NG + b],
            )

        for b in range(NG):
            g_copy(0, b, b).start()

        def pair_body(r2, carry):
            ca = 2 * r2 * NG
            cb = ca + NG
            for b in range(NG):
                g_copy(0, b, ca + b).wait()
                s_copy(0, b, ca + b).start()
            for b in range(NG):
                @pl.when(r2 > 0)
                def _():
                    s_copy(1, b, cb + b - 2 * NG).wait()
                g_copy(1, b, cb + b).start()
            for b in range(NG):
                g_copy(1, b, cb + b).wait()
                s_copy(1, b, cb + b).start()
            for b in range(NG):
                s_copy(0, b, ca + b).wait()
                @pl.when(r2 < pairs - 1)
                def _():
                    g_copy(0, b, ca + b + 2 * NG).start()
            return carry

        lax.fori_loop(0, pairs, pair_body, 0)

        last_cb = (2 * (pairs - 1) + 1) * NG
        for b in range(NG):
            s_copy(1, b, last_cb + b).wait()

    return gather_kernel


def kernel(idx, logits_table):
    b, t = idx.shape
    table2 = logits_table.reshape(VOCAB * SPLIT, D2)
    idxf = idx.reshape(-1).astype(jnp.int32)
    idx2 = (idxf[:, None] * SPLIT
            + jnp.arange(SPLIT, dtype=jnp.int32)[None, :]).reshape(-1)
    out2 = _make_gather(idx2.shape[0])(table2, idx2)
    return out2.reshape(b, t, D)


# speedup vs baseline: 3.8627x; 3.7672x over previous
import functools

import jax
import jax.numpy as jnp
from jax import lax
from jax.experimental import pallas as pl
from jax.experimental.pallas import tpu as pltpu
from jax.experimental.pallas import tpu_sc as plsc

VOCAB = 8192
D = 8192
NC = 2
NS = 16
NW = NC * NS
K = 2
NG = 2


def _make_gather(n):
    pw = n // NW
    chunks = pw // K
    rounds = chunks // NG
    pairs = rounds // 2
    mesh = plsc.VectorSubcoreMesh(core_axis_name="c", subcore_axis_name="s")

    @functools.partial(
        pl.kernel,
        mesh=mesh,
        out_type=jax.ShapeDtypeStruct((n, D), jnp.float32),
        scratch_types=[
            pltpu.VMEM((chunks, K), jnp.int32),
            [pltpu.VMEM((K, D), jnp.float32)] * (2 * NG),
            [pltpu.SemaphoreType.DMA] * (2 * NG),
            [pltpu.SemaphoreType.DMA] * (2 * NG),
        ],
    )
    def gather_kernel(table_hbm, idx_hbm, out_hbm, idx_v, rows_v, gsems, ssems):
        cid = lax.axis_index("c")
        sid = lax.axis_index("s")
        wid = sid * NC + cid
        base = wid * pw
        pltpu.sync_copy(idx_hbm.at[wid], idx_v)

        def buf(g, b):
            return rows_v[g * NG + b]

        def g_copy(g, b, c):
            return pltpu.make_async_copy(
                table_hbm.at[idx_v.at[c]],
                buf(g, b),
                gsems[g * NG + b],
            )

        def s_copy(g, b, c):
            return pltpu.make_async_copy(
                buf(g, b),
                out_hbm.at[pl.ds(base + c * K, K)],
                ssems[g * NG + b],
            )

        for b in range(NG):
            g_copy(0, b, b).start()

        def pair_body(r2, carry):
            ca = 2 * r2 * NG
            cb = ca + NG
            for b in range(NG):
                g_copy(0, b, ca + b).wait()
                s_copy(0, b, ca + b).start()
            for b in range(NG):
                @pl.when(r2 > 0)
                def _():
                    s_copy(1, b, cb + b - 2 * NG).wait()
                g_copy(1, b, cb + b).start()
            for b in range(NG):
                g_copy(1, b, cb + b).wait()
                s_copy(1, b, cb + b).start()
            for b in range(NG):
                s_copy(0, b, ca + b).wait()
                @pl.when(r2 < pairs - 1)
                def _():
                    g_copy(0, b, ca + b + 2 * NG).start()
            return carry

        lax.fori_loop(0, pairs, pair_body, 0)

        last_cb = (2 * (pairs - 1) + 1) * NG
        for b in range(NG):
            s_copy(1, b, last_cb + b).wait()

    return gather_kernel


def kernel(idx, logits_table):
    b, t = idx.shape
    n = b * t
    idx3 = idx.reshape(NW, (n // NW) // K, K).astype(jnp.int32)
    out2 = _make_gather(n)(logits_table, idx3)
    return out2.reshape(b, t, D)
